# Initial kernel scaffold; baseline (speedup 1.0000x reference)
import jax
import jax.numpy as jnp
from jax.experimental import pallas as pl


def kernel(InState, NNsites, GnnPerms, SitesToShells, gdiags, Psi0, b0, Psi1, b1, Psi2, b2, Psi3, b3, Psi4, b4, PsiR, VR):
    raise NotImplementedError("write your pallas kernel here")


# speedup vs baseline: 13.8225x; 13.8225x over previous
import functools

import jax
import jax.numpy as jnp
from jax import lax
from jax.experimental import pallas as pl
from jax.experimental.pallas import tpu as pltpu
from jax.experimental.pallas import tpu_sc as plsc

_B = 4
_C = 8
_S = 8192
_N = 13
_G = 48
_K = 6
_D = 8
_NROWS = _B * _S
_NIDX = _B * _S * _N
_NC, _NS = 2, 16
_NW = _NC * _NS
_IPW = _NIDX // _NW


def _make_sc_gather():
    mesh = plsc.VectorSubcoreMesh(core_axis_name="c", subcore_axis_name="s")

    @functools.partial(
        pl.kernel,
        out_type=jax.ShapeDtypeStruct((_NIDX, _D), jnp.float32),
        mesh=mesh,
        scratch_types=[
            pltpu.VMEM((_IPW,), jnp.int32),
            pltpu.VMEM((_IPW, _D), jnp.float32),
            pltpu.SemaphoreType.DMA,
        ],
        compiler_params=pltpu.CompilerParams(use_tc_tiling_on_sc=False),
    )
    def gather_k(table_hbm, idx_hbm, out_hbm, idx_v, rows_v, sem):
        wid = lax.axis_index("s") * _NC + lax.axis_index("c")
        base = wid * _IPW
        pltpu.sync_copy(idx_hbm.at[pl.ds(base, _IPW)], idx_v)
        pltpu.async_copy(table_hbm.at[idx_v], rows_v, sem).wait()
        pltpu.sync_copy(rows_v, out_hbm.at[pl.ds(base, _IPW)])

    return gather_k


def _layer_body(r_ref, w_ref, b_ref, m_ref, o_ref):
    x = jnp.dot(r_ref[...], w_ref[...], preferred_element_type=jnp.float32, precision=lax.Precision.HIGHEST)
    x = x + b_ref[...]
    sp = jnp.maximum(x, 0.0) + jnp.log(1.0 + jnp.exp(-jnp.abs(x)))
    o_ref[...] = jnp.dot(sp, m_ref[...], preferred_element_type=jnp.float32, precision=lax.Precision.HIGHEST)


def _layer_tc(rows, w, bias, mavg, bz=2048):
    z, k = rows.shape
    go = w.shape[1]
    return pl.pallas_call(
        _layer_body,
        grid=(z // bz,),
        in_specs=[
            pl.BlockSpec((bz, k), lambda i: (i, 0)),
            pl.BlockSpec((k, go), lambda i: (0, 0)),
            pl.BlockSpec((1, go), lambda i: (0, 0)),
            pl.BlockSpec((go, _D), lambda i: (0, 0)),
        ],
        out_specs=pl.BlockSpec((bz, _D), lambda i: (i, 0)),
        out_shape=jax.ShapeDtypeStruct((z, _D), jnp.float32),
    )(rows, w, bias, mavg)


def _final_body(r_ref, sh_ref, b1_ref, k1_ref, d1_ref, o_ref):
    b = pl.program_id(0)
    j = pl.program_id(1)
    sh = sh_ref[...]
    kio = lax.broadcasted_iota(jnp.int32, (1, _K), 1)
    mask = (sh == kio).astype(jnp.float32)
    mask2 = jnp.dot(mask, b1_ref[...], preferred_element_type=jnp.float32, precision=lax.Precision.HIGHEST)
    c1 = jnp.dot(r_ref[...], k1_ref[...], preferred_element_type=jnp.float32, precision=lax.Precision.HIGHEST)
    p = jnp.dot(mask2 * c1, d1_ref[...], preferred_element_type=jnp.float32, precision=lax.Precision.HIGHEST)
    part = jnp.sum(p, axis=0, keepdims=True)

    @pl.when((b == 0) & (j == 0))
    def _():
        o_ref[...] = jnp.zeros_like(o_ref)

    o_ref[pl.ds(b, 1), :] += part


def _final_tc(rows, shells2d, b1, k1, d1, bz=2048):
    nsb = _S // bz
    kg = k1.shape[1]
    return pl.pallas_call(
        _final_body,
        grid=(_B, nsb),
        in_specs=[
            pl.BlockSpec((bz, _N * _D), lambda b, j: (b * nsb + j, 0)),
            pl.BlockSpec((bz, 1), lambda b, j: (b * nsb + j, 0)),
            pl.BlockSpec((_K, kg), lambda b, j: (0, 0)),
            pl.BlockSpec((_N * _D, kg), lambda b, j: (0, 0)),
            pl.BlockSpec((kg, 3), lambda b, j: (0, 0)),
        ],
        out_specs=pl.BlockSpec((_B, 3), lambda b, j: (0, 0)),
        out_shape=jax.ShapeDtypeStruct((_B, 3), jnp.float32),
    )(rows, shells2d, b1, k1, d1)


def kernel(InState, NNsites, GnnPerms, SitesToShells, gdiags,
           Psi0, b0, Psi1, b1, Psi2, b2, Psi3, b3, Psi4, b4, PsiR, VR):
    f32 = jnp.float32
    x = jnp.transpose(InState, (0, 2, 1))
    x = jnp.pad(x, ((0, 0), (0, 0), (0, _D - x.shape[2])))
    table = x.reshape(_NROWS, _D)

    idx = (jnp.arange(_B, dtype=jnp.int32)[:, None, None] * _S
           + NNsites.T[None, :, :]).reshape(_NIDX)

    ws, bs, ms = [], [], []
    for psi, bias in ((Psi0, b0), (Psi1, b1), (Psi2, b2), (Psi3, b3), (Psi4, b4)):
        o, cin, _ = psi.shape
        psip = jnp.pad(psi, ((0, 0), (0, _C - cin), (0, 0)))
        psig = psip[:, :, GnnPerms]
        w = jnp.transpose(psig, (3, 1, 2, 0)).reshape(_N * _C, _G * o)
        ws.append(w.astype(f32))
        bs.append(jnp.tile(bias, _G)[None, :].astype(f32))
        ms.append((jnp.tile(jnp.eye(o, _D), (_G, 1)) / _G).astype(f32))

    sc_gather = _make_sc_gather()
    for l in range(5):
        rows = sc_gather(table, idx).reshape(_NROWS, _N * _D)
        table = _layer_tc(rows, ws[l], bs[l], ms[l])

    rows = sc_gather(table, idx).reshape(_NROWS, _N * _D)
    psirg = PsiR[:, GnnPerms]
    k1 = jnp.zeros((_N, _C, _K * _G), f32)
    k1 = k1.at[:, 0, :].set(jnp.transpose(psirg, (2, 0, 1)).reshape(_N, _K * _G))
    k1 = k1.reshape(_N * _C, _K * _G)
    b1m = jnp.kron(jnp.eye(_K, dtype=f32), jnp.ones((1, _G), f32))
    d1 = (jnp.einsum('kd,gde->kge', VR, gdiags) / _G).reshape(_K * _G, 3)
    shells2d = jnp.tile(SitesToShells.astype(jnp.int32), _B)[:, None]
    return _final_tc(rows, shells2d, b1m, k1, d1.astype(f32))
